# baseline (device time: 50950 ns/iter reference)
import jax
import jax.numpy as jnp
from jax import lax
from jax.experimental import pallas as pl
from jax.experimental.pallas import tpu as pltpu

N_DEV = 8
M = 1024

NS = 8
RB = M // NS
BANDS = tuple((s * RB, RB) for s in range(NS))
LS = ((0, 1, 2), (1, 2, 0), (2, 0, 1))
PERMS = tuple(LS[s % 3] for s in range(NS))

OPS = (
    ("rs0", "rs1", "fe2", "ag1", "ag0"),
    ("rs0", "rs1", "fe2", "ag1", "ag0"),
    ("rs0", "rs1", "fe2", "ag1", "ag0"),
    ("rs0", "rs1", "fe2", "ag1", "ag0"),
    ("rs0", "fe1", "fe2", "ag0"),
    ("rs0", "fe1", "fe2", "ag0"),
    ("fe0", "fe1", "fe2"),
    ("fe0", "fe1", "fe2"),
)
MAX_STEPS = 5

STAGE_OFFS = (
    (0, 64, 96),
    (0, 64, 96),
    (0, 64, 96),
    (0, 64, 96),
    (0, 64, 128),
    (0, 64, 128),
    (0, 128, 256),
    (0, 128, 256),
)
STAGE_SIZES = (128, 128, 128, 128, 192, 192, 384, 384)
STAGE_BASE = tuple(sum(STAGE_SIZES[:s]) for s in range(NS))
COMM_ROWS = sum(STAGE_SIZES)

MM_CHUNK = 256


def _id_to_bits(my):
    p = lax.rem(my, 4)
    z = my // 4
    y = p // 2
    x = lax.rem((p + 1) // 2, 2)
    return (x, y, z)


def _bits_to_id(x, y, z):
    return 4 * z + 2 * y + (x ^ y)


def kernel(dy, W):
    m, k = dy.shape
    assert m == M

    def body(dy_ref, w_ref, out_ref, acc_ref, w16_ref, comm_ref, send_sems, recv_sems):
        my = lax.axis_index("i")
        x, y, z = _id_to_bits(my)
        bits = (x, y, z)
        partner = (
            _bits_to_id(1 - x, y, z),
            _bits_to_id(x, 1 - y, z),
            _bits_to_id(x, y, 1 - z),
        )

        w16_ref[...] = w_ref[...].astype(jnp.bfloat16)

        barrier_sem = pltpu.get_barrier_semaphore()
        for d in range(3):
            pl.semaphore_signal(
                barrier_sem, inc=1,
                device_id=(partner[d],), device_id_type=pl.DeviceIdType.MESH,
            )
        pl.semaphore_wait(barrier_sem, 3)

        off = [None] * NS
        size = [None] * NS
        pend = [None] * NS

        def start_op(s, j):
            op = OPS[s][j]
            kind, ph = op[:2], int(op[2])
            d = PERMS[s][ph]
            b = bits[d]
            sems = (send_sems.at[s * 6 + j], recv_sems.at[s * 6 + j])
            if kind == "rs":
                half = size[s] // 2
                stage = STAGE_BASE[s] + STAGE_OFFS[s][j]
                keep_off = off[s] + b * half
                send_off = off[s] + (1 - b) * half
                rdma = pltpu.make_async_remote_copy(
                    src_ref=acc_ref.at[pl.ds(send_off, half), :],
                    dst_ref=comm_ref.at[pl.ds(stage, half), :],
                    send_sem=sems[0], recv_sem=sems[1],
                    device_id=(partner[d],),
                    device_id_type=pl.DeviceIdType.MESH,
                )
                rdma.start()
                pend[s] = (rdma, keep_off, half, stage)
                off[s] = keep_off
                size[s] = half
            elif kind == "fe":
                stage = STAGE_BASE[s] + STAGE_OFFS[s][j]
                rdma = pltpu.make_async_remote_copy(
                    src_ref=acc_ref.at[pl.ds(off[s], size[s]), :],
                    dst_ref=comm_ref.at[pl.ds(stage, size[s]), :],
                    send_sem=sems[0], recv_sem=sems[1],
                    device_id=(partner[d],),
                    device_id_type=pl.DeviceIdType.MESH,
                )
                rdma.start()
                pend[s] = (rdma, off[s], size[s], stage)
            else:
                rdma = pltpu.make_async_remote_copy(
                    src_ref=acc_ref.at[pl.ds(off[s], size[s]), :],
                    dst_ref=acc_ref.at[pl.ds(off[s], size[s]), :],
                    send_sem=sems[0], recv_sem=sems[1],
                    device_id=(partner[d],),
                    device_id_type=pl.DeviceIdType.MESH,
                )
                rdma.start()
                pend[s] = (rdma, None, None, None)
                off[s] = off[s] - b * size[s]
                size[s] = size[s] * 2

        def finish(s):
            rdma, add_off, rows, stage = pend[s]
            rdma.wait()
            if add_off is not None:
                acc_ref[pl.ds(add_off, rows), :] = (
                    acc_ref[pl.ds(add_off, rows), :]
                    + comm_ref[pl.ds(stage, rows), :]
                )

        for c in range(M // MM_CHUNK):
            B = c * MM_CHUNK
            acc_ref[pl.ds(B, MM_CHUNK), :] = lax.dot_general(
                dy_ref[pl.ds(B, MM_CHUNK), :].astype(jnp.bfloat16), w16_ref[...],
                dimension_numbers=(((1,), (1,)), ((), ())),
                preferred_element_type=jnp.float32,
            ).astype(jnp.bfloat16)
            for s in range(B // RB, (B + MM_CHUNK) // RB):
                off[s] = BANDS[s][0]
                size[s] = BANDS[s][1]
                start_op(s, 0)

        for r in range(1, MAX_STEPS):
            for s in range(NS):
                if r < len(OPS[s]):
                    finish(s)
                    start_op(s, r)
        for s in range(NS):
            finish(s)
            B, R = BANDS[s]
            out_ref[pl.ds(B, R), :] = acc_ref[pl.ds(B, R), :].astype(jnp.float32)

    return pl.pallas_call(
        body,
        out_shape=jax.ShapeDtypeStruct((M, M), jnp.float32),
        in_specs=[
            pl.BlockSpec(memory_space=pltpu.VMEM),
            pl.BlockSpec(memory_space=pltpu.VMEM),
        ],
        out_specs=pl.BlockSpec(memory_space=pltpu.VMEM),
        scratch_shapes=[
            pltpu.VMEM((M, M), jnp.bfloat16),
            pltpu.VMEM((M, 4096), jnp.bfloat16),
            pltpu.VMEM((COMM_ROWS, M), jnp.bfloat16),
            pltpu.SemaphoreType.DMA((NS * 6,)),
            pltpu.SemaphoreType.DMA((NS * 6,)),
        ],
        compiler_params=pltpu.CompilerParams(
            collective_id=0, vmem_limit_bytes=60 * 1024 * 1024
        ),
    )(dy, W)


# device time: 42577 ns/iter; 1.1967x vs baseline; 1.1967x over previous
import jax
import jax.numpy as jnp
from jax import lax
from jax.experimental import pallas as pl
from jax.experimental.pallas import tpu as pltpu

N_DEV = 8
M = 1024

NS = 8
RB = M // NS
BANDS = tuple((s * RB, RB) for s in range(NS))
LS = ((0, 1, 2), (1, 2, 0), (2, 0, 1))
PERMS = tuple(LS[s % 3] for s in range(NS))
STAGE_ROWS = RB
STAGE_BASE = tuple(s * STAGE_ROWS for s in range(NS))
COMM_ROWS = NS * STAGE_ROWS

MM_CHUNK = 256


def _id_to_bits(my):
    p = lax.rem(my, 4)
    z = my // 4
    y = p // 2
    x = lax.rem((p + 1) // 2, 2)
    return (x, y, z)


def _bits_to_id(x, y, z):
    return 4 * z + 2 * y + (x ^ y)


def kernel(dy, W):
    m, k = dy.shape
    assert m == M

    def body(dy_ref, w_ref, out_ref, acc_ref, w16_ref, dyv_ref, wv_ref,
             comm_ref, load_sems, send_sems, recv_sems):
        my = lax.axis_index("i")
        x, y, z = _id_to_bits(my)
        bits = (x, y, z)
        partner = (
            _bits_to_id(1 - x, y, z),
            _bits_to_id(x, 1 - y, z),
            _bits_to_id(x, y, 1 - z),
        )

        w_cp = pltpu.make_async_copy(w_ref, wv_ref, load_sems.at[0])
        w_cp.start()
        dy_cps = []
        for c in range(M // MM_CHUNK):
            B = c * MM_CHUNK
            cp = pltpu.make_async_copy(
                dy_ref.at[pl.ds(B, MM_CHUNK), :],
                dyv_ref.at[pl.ds(B, MM_CHUNK), :],
                load_sems.at[1 + c],
            )
            cp.start()
            dy_cps.append(cp)

        barrier_sem = pltpu.get_barrier_semaphore()
        for d in range(3):
            pl.semaphore_signal(
                barrier_sem, inc=1,
                device_id=(partner[d],), device_id_type=pl.DeviceIdType.MESH,
            )
        pl.semaphore_wait(barrier_sem, 3)

        w_cp.wait()
        w16_ref[...] = wv_ref[...].astype(jnp.bfloat16)

        off = [None] * NS
        size = [None] * NS
        pend = [None] * NS

        def start_rs(s, ph):
            d = PERMS[s][ph]
            b = bits[d]
            half = size[s] // 2
            stage = STAGE_BASE[s] + (0, RB // 2, 3 * RB // 4)[ph]
            keep_off = off[s] + b * half
            send_off = off[s] + (1 - b) * half
            rdma = pltpu.make_async_remote_copy(
                src_ref=acc_ref.at[pl.ds(send_off, half), :],
                dst_ref=comm_ref.at[pl.ds(stage, half), :],
                send_sem=send_sems.at[s * 6 + ph],
                recv_sem=recv_sems.at[s * 6 + ph],
                device_id=(partner[d],),
                device_id_type=pl.DeviceIdType.MESH,
            )
            rdma.start()
            pend[s] = (rdma, keep_off, half, stage)
            off[s] = keep_off
            size[s] = half

        def start_fe(s, ph):
            d = PERMS[s][ph]
            stage = STAGE_BASE[s] + 3 * RB // 4
            rdma = pltpu.make_async_remote_copy(
                src_ref=acc_ref.at[pl.ds(off[s], size[s]), :],
                dst_ref=comm_ref.at[pl.ds(stage, size[s]), :],
                send_sem=send_sems.at[s * 6 + ph],
                recv_sem=recv_sems.at[s * 6 + ph],
                device_id=(partner[d],),
                device_id_type=pl.DeviceIdType.MESH,
            )
            rdma.start()
            pend[s] = (rdma, off[s], size[s], stage)

        def start_ag(s, ph):
            d = PERMS[s][ph]
            b = bits[d]
            rdma = pltpu.make_async_remote_copy(
                src_ref=acc_ref.at[pl.ds(off[s], size[s]), :],
                dst_ref=acc_ref.at[pl.ds(off[s], size[s]), :],
                send_sem=send_sems.at[s * 6 + 3 + ph],
                recv_sem=recv_sems.at[s * 6 + 3 + ph],
                device_id=(partner[d],),
                device_id_type=pl.DeviceIdType.MESH,
            )
            rdma.start()
            pend[s] = (rdma, None, None, None)
            off[s] = off[s] - b * size[s]
            size[s] = size[s] * 2

        def finish(s):
            rdma, keep_off, half, stage = pend[s]
            rdma.wait()
            if keep_off is not None:
                acc_ref[pl.ds(keep_off, half), :] = (
                    acc_ref[pl.ds(keep_off, half), :]
                    + comm_ref[pl.ds(stage, half), :]
                )

        for c in range(M // MM_CHUNK):
            B = c * MM_CHUNK
            dy_cps[c].wait()
            acc_ref[pl.ds(B, MM_CHUNK), :] = lax.dot_general(
                dyv_ref[pl.ds(B, MM_CHUNK), :].astype(jnp.bfloat16), w16_ref[...],
                dimension_numbers=(((1,), (1,)), ((), ())),
                preferred_element_type=jnp.float32,
            ).astype(jnp.bfloat16)
            for s in range(B // RB, (B + MM_CHUNK) // RB):
                off[s] = jnp.int32(BANDS[s][0])
                size[s] = BANDS[s][1]
                start_rs(s, 0)

        for step in range(1, 5):
            for s in range(NS):
                finish(s)
                if step == 1:
                    start_rs(s, 1)
                elif step == 2:
                    start_fe(s, 2)
                else:
                    start_ag(s, 4 - step)
        for s in range(NS):
            finish(s)
            B, R = BANDS[s]
            out_ref[pl.ds(B, R), :] = acc_ref[pl.ds(B, R), :].astype(jnp.float32)

    return pl.pallas_call(
        body,
        out_shape=jax.ShapeDtypeStruct((M, M), jnp.float32),
        in_specs=[
            pl.BlockSpec(memory_space=pl.ANY),
            pl.BlockSpec(memory_space=pl.ANY),
        ],
        out_specs=pl.BlockSpec(memory_space=pltpu.VMEM),
        scratch_shapes=[
            pltpu.VMEM((M, M), jnp.bfloat16),
            pltpu.VMEM((M, 4096), jnp.bfloat16),
            pltpu.VMEM((M, 4096), jnp.float32),
            pltpu.VMEM((M, 4096), jnp.float32),
            pltpu.VMEM((COMM_ROWS, M), jnp.bfloat16),
            pltpu.SemaphoreType.DMA((5,)),
            pltpu.SemaphoreType.DMA((NS * 6,)),
            pltpu.SemaphoreType.DMA((NS * 6,)),
        ],
        compiler_params=pltpu.CompilerParams(
            collective_id=0, vmem_limit_bytes=60 * 1024 * 1024
        ),
    )(dy, W)
